# baseline (speedup 1.0000x reference)
import functools

import jax
import jax.numpy as jnp
from jax import lax
from jax.experimental import pallas as pl
from jax.experimental.pallas import tpu as pltpu
from jax.experimental.pallas import tpu_sc as plsc

N = 10000
NP = 10240
D = 128
E = 320000
OUT = 10
GRAPHS = 64
HID = 128
NC, NS = 2, 16
NW = NC * NS
CH = 80
ROWS2D = NW * CH
EPAD = ROWS2D * 128
RPT = NP // NS
RB = 512
GRID = NP // RB
HP = lax.Precision.HIGHEST

_MESH = dict(core_axis_name="c", subcore_axis_name="s",
             num_cores=NC, num_subcores=NS)


def _sc_degree_body(dst2d, out, didx, ones_v, zrow, hist):
    c = lax.axis_index("c")
    s = lax.axis_index("s")
    w = s * NC + c

    def fill_zero(i, carry):
        zrow[pl.ds(i * 16, 16)] = jnp.zeros((16,), jnp.float32)
        return carry

    lax.fori_loop(0, RPT // 16, fill_zero, 0)
    for k in range(8):
        ones_v[pl.ds(k * 16, 16)] = jnp.ones((16,), jnp.float32)
    pltpu.sync_copy(zrow, hist.at[pl.ds(s * RPT, RPT)])
    pltpu.sync_copy(dst2d.at[pl.ds(w * CH, CH)], didx)
    plsc.subcore_barrier()

    def body(j, carry):
        pltpu.sync_copy(ones_v, hist.at[didx.at[j]], add=True)
        return carry

    lax.fori_loop(0, CH, body, 0)
    plsc.subcore_barrier()
    pltpu.sync_copy(hist.at[pl.ds(s * RPT, RPT)], out.at[c, pl.ds(s * RPT, RPT)])


CH0 = 160
SEGA = 32
NSEGA = CH0 // SEGA


def _sc_agg_body(g, src2d, dst2d, out, sa, sb, dseg, r0, r1, acc, s0, s1):
    c = lax.axis_index("c")
    s = lax.axis_index("s")
    bufs = (r0, r1)
    sems = (s0, s1)
    ssegs = (sa, sb)
    base = s * CH0

    def run_seg(t, issue_next):
        cur = ssegs[t % 2]
        nxt = ssegs[(t + 1) % 2]
        pltpu.sync_copy(dst2d.at[pl.ds(base + t * SEGA, SEGA)], dseg)
        if issue_next:
            pltpu.sync_copy(src2d.at[pl.ds(base + (t + 1) * SEGA, SEGA)], nxt)

        def pair(p, carry):
            for k in range(2):
                j = p * 2 + k
                pltpu.make_async_copy(g.at[cur.at[j]], bufs[k], sems[k]).wait()
                pltpu.sync_copy(bufs[k], acc.at[dseg.at[j]], add=True)
                pltpu.async_copy(g.at[cur.at[j + 2]], bufs[k], sems[k])
            return carry

        lax.fori_loop(0, SEGA // 2 - 1, pair, 0)
        for k in range(2):
            j = SEGA - 2 + k
            pltpu.make_async_copy(g.at[cur.at[j]], bufs[k], sems[k]).wait()
            pltpu.sync_copy(bufs[k], acc.at[dseg.at[j]], add=True)
            if issue_next:
                pltpu.async_copy(g.at[nxt.at[k]], bufs[k], sems[k])
            else:
                pltpu.async_copy(g.at[cur.at[SEGA - 1]], bufs[k], sems[k])

    @pl.when(c == 0)
    def _():
        def zfill(i, carry):
            r0[i // 8, pl.ds((i % 8) * 16, 16)] = jnp.zeros((16,), jnp.float32)
            return carry

        lax.fori_loop(0, 128 * 8, zfill, 0)
        for t in range(RPT // 128):
            pltpu.sync_copy(r0, acc.at[pl.ds(s * RPT + t * 128, 128)])
        pltpu.sync_copy(src2d.at[pl.ds(base, SEGA)], sa)
        plsc.subcore_barrier()

        pltpu.async_copy(g.at[sa.at[0]], r0, s0)
        pltpu.async_copy(g.at[sa.at[1]], r1, s1)

        for t in range(NSEGA):
            run_seg(t, t < NSEGA - 1)

        pltpu.make_async_copy(g.at[sa.at[0]], r0, s0).wait()
        pltpu.make_async_copy(g.at[sa.at[1]], r1, s1).wait()
        plsc.subcore_barrier()
        pltpu.sync_copy(acc.at[pl.ds(s * RPT, RPT)],
                        out.at[0, pl.ds(s * RPT, RPT)])


@functools.lru_cache(maxsize=None)
def _sc_kernels():
    mesh = plsc.VectorSubcoreMesh(**_MESH)
    sc_degree = pl.kernel(
        _sc_degree_body,
        out_type=jax.ShapeDtypeStruct((NC, NP), jnp.float32),
        mesh=mesh,
        scratch_types=[
            pltpu.VMEM((CH, 128), jnp.int32),
            pltpu.VMEM((128,), jnp.float32),
            pltpu.VMEM((RPT,), jnp.float32),
            pltpu.VMEM_SHARED((NP,), jnp.float32),
        ],
    )
    sc_agg = pl.kernel(
        _sc_agg_body,
        out_type=jax.ShapeDtypeStruct((1, NP, D), jnp.float32),
        mesh=mesh,
        scratch_types=[
            pltpu.VMEM((SEGA, 128), jnp.int32),
            pltpu.VMEM((SEGA, 128), jnp.int32),
            pltpu.VMEM((SEGA, 128), jnp.int32),
            pltpu.VMEM((128, D), jnp.float32),
            pltpu.VMEM((128, D), jnp.float32),
            pltpu.VMEM_SHARED((NP, D), jnp.float32),
            pltpu.SemaphoreType.DMA,
            pltpu.SemaphoreType.DMA,
        ],
    )
    return sc_degree, sc_agg


def _dinv(deg_ref):
    deg = deg_ref[0, :] + deg_ref[1, :] + 1.0
    return 1.0 / jnp.sqrt(deg)


def _tc_g1(deg_ref, x_ref, w_ref, o_ref):
    dinv = _dinv(deg_ref)
    h = lax.dot_general(x_ref[...], w_ref[...], (((1,), (0,)), ((), ())),
                        precision=HP)
    o_ref[...] = h * dinv[:, None]


def _tc_mid(deg_ref, agg_ref, g_ref, w_ref, b_ref, o_ref):
    dinv = _dinv(deg_ref)
    tot = agg_ref[0] + g_ref[...]
    h1 = jnp.maximum(tot * dinv[:, None] + b_ref[0, :][None, :], 0.0)
    h2 = lax.dot_general(h1, w_ref[...], (((1,), (0,)), ((), ())), precision=HP)
    o_ref[...] = h2 * dinv[:, None]


def _tc_final(deg_ref, agg_ref, g_ref, b2_ref, batch_ref, wv_ref, bv_ref,
              wo_ref, bo_ref, fw_ref, fb_ref, out_ref, acc, cnt):
    i = pl.program_id(0)
    dinv = _dinv(deg_ref)
    tot = agg_ref[0] + g_ref[...]
    h = jnp.maximum(tot * dinv[:, None] + b2_ref[0, :][None, :], 0.0)
    b = batch_ref[0, 0, :]
    gi = lax.broadcasted_iota(jnp.int32, (GRAPHS, RB), 0)
    oh = (gi == b[None, :]).astype(jnp.float32)
    ps = lax.dot_general(oh, h, (((1,), (0,)), ((), ())), precision=HP)
    pc = jnp.broadcast_to(jnp.sum(oh, axis=1)[:, None], (GRAPHS, D))

    @pl.when(i == 0)
    def _():
        acc[...] = ps
        cnt[...] = pc

    @pl.when(i > 0)
    def _():
        acc[...] += ps
        cnt[...] += pc

    @pl.when(i == GRID - 1)
    def _():
        cvals = cnt[...]
        mean = acc[...] / jnp.maximum(cvals, 1.0)
        t = lax.dot_general(mean, wv_ref[...], (((1,), (0,)), ((), ())),
                            precision=HP)
        t = lax.dot_general(t, wo_ref[...], (((1,), (0,)), ((), ())),
                            precision=HP)
        bias2 = lax.dot_general(bv_ref[...], wo_ref[...],
                                (((1,), (0,)), ((), ())), precision=HP) + bo_ref[...]
        nz = jnp.where(cvals > 0.0, 1.0, 0.0)
        h2p = t + nz * bias2
        res = lax.dot_general(h2p, fw_ref[...], (((1,), (0,)), ((), ())),
                              precision=HP) + fb_ref[...]
        out_ref[...] = res


def kernel(x, edge_index, batch, W1, b1, W2, b2, in_proj_w, in_proj_b,
           out_proj_w, out_proj_b, fc_w, fc_b):
    f32 = jnp.float32
    src, dst = edge_index[0], edge_index[1]
    pad_e = EPAD - E
    src2d = jnp.concatenate(
        [src, jnp.zeros((pad_e,), jnp.int32)]).reshape(ROWS2D, 128)
    dst2d = jnp.concatenate(
        [dst, jnp.full((pad_e,), NP - 1, jnp.int32)]).reshape(ROWS2D, 128)
    xp = jnp.zeros((NP, D), f32).at[:N].set(x)
    batch3d = jnp.concatenate(
        [batch, jnp.full((NP - N,), GRAPHS, jnp.int32)]).reshape(GRID, 1, RB)

    _sc_degree, _sc_agg = _sc_kernels()
    degp = _sc_degree(dst2d)

    g1 = pl.pallas_call(
        _tc_g1, grid=(GRID,),
        in_specs=[pl.BlockSpec((2, RB), lambda i: (0, i)),
                  pl.BlockSpec((RB, D), lambda i: (i, 0)),
                  pl.BlockSpec((D, D), lambda i: (0, 0))],
        out_specs=pl.BlockSpec((RB, D), lambda i: (i, 0)),
        out_shape=jax.ShapeDtypeStruct((NP, D), f32),
    )(degp, xp, W1)

    agg1 = _sc_agg(g1, src2d, dst2d)

    g2 = pl.pallas_call(
        _tc_mid, grid=(GRID,),
        in_specs=[pl.BlockSpec((2, RB), lambda i: (0, i)),
                  pl.BlockSpec((1, RB, D), lambda i: (0, i, 0)),
                  pl.BlockSpec((RB, D), lambda i: (i, 0)),
                  pl.BlockSpec((D, D), lambda i: (0, 0)),
                  pl.BlockSpec((1, D), lambda i: (0, 0))],
        out_specs=pl.BlockSpec((RB, D), lambda i: (i, 0)),
        out_shape=jax.ShapeDtypeStruct((NP, D), f32),
    )(degp, agg1, g1, W2, b1.reshape(1, D))

    agg2 = _sc_agg(g2, src2d, dst2d)

    res = pl.pallas_call(
        _tc_final, grid=(GRID,),
        in_specs=[pl.BlockSpec((2, RB), lambda i: (0, i)),
                  pl.BlockSpec((1, RB, D), lambda i: (0, i, 0)),
                  pl.BlockSpec((RB, D), lambda i: (i, 0)),
                  pl.BlockSpec((1, D), lambda i: (0, 0)),
                  pl.BlockSpec((1, 1, RB), lambda i: (i, 0, 0)),
                  pl.BlockSpec((D, D), lambda i: (0, 0)),
                  pl.BlockSpec((1, D), lambda i: (0, 0)),
                  pl.BlockSpec((D, D), lambda i: (0, 0)),
                  pl.BlockSpec((1, D), lambda i: (0, 0)),
                  pl.BlockSpec((D, D), lambda i: (0, 0)),
                  pl.BlockSpec((1, D), lambda i: (0, 0))],
        out_specs=pl.BlockSpec((GRAPHS, D), lambda i: (0, 0)),
        out_shape=jax.ShapeDtypeStruct((GRAPHS, D), f32),
        scratch_shapes=[pltpu.VMEM((GRAPHS, D), f32),
                        pltpu.VMEM((GRAPHS, D), f32)],
    )(degp, agg2, g2, b2.reshape(1, D), batch3d,
      in_proj_w[2 * HID:].T, in_proj_b[2 * HID:].reshape(1, D),
      out_proj_w.T, out_proj_b.reshape(1, D),
      jnp.zeros((D, D), f32).at[:, :OUT].set(fc_w.T),
      jnp.zeros((1, D), f32).at[0, :OUT].set(fc_b))

    return res[:, :OUT]


# speedup vs baseline: 4.0111x; 4.0111x over previous
import functools

import jax
import jax.numpy as jnp
from jax import lax
from jax.experimental import pallas as pl
from jax.experimental.pallas import tpu as pltpu
from jax.experimental.pallas import tpu_sc as plsc

N = 10000
NP = 10240
D = 128
E = 320000
OUT = 10
GRAPHS = 64
HID = 128
NC, NS = 2, 16
NW = NC * NS
CH = 80
ROWS2D = NW * CH
EPAD = ROWS2D * 128
RPT = NP // NS
RB = 512
GRID = NP // RB
HP = lax.Precision.HIGHEST

_MESH = dict(core_axis_name="c", subcore_axis_name="s",
             num_cores=NC, num_subcores=NS)


def _sc_degree_body(dst2d, out, didx, ones_v, zrow, hist):
    c = lax.axis_index("c")
    s = lax.axis_index("s")
    w = s * NC + c

    def fill_zero(i, carry):
        zrow[pl.ds(i * 16, 16)] = jnp.zeros((16,), jnp.float32)
        return carry

    lax.fori_loop(0, RPT // 16, fill_zero, 0)
    for k in range(8):
        ones_v[pl.ds(k * 16, 16)] = jnp.ones((16,), jnp.float32)
    pltpu.sync_copy(zrow, hist.at[pl.ds(s * RPT, RPT)])
    pltpu.sync_copy(dst2d.at[pl.ds(w * CH, CH)], didx)
    plsc.subcore_barrier()

    def body(j, carry):
        pltpu.sync_copy(ones_v, hist.at[didx.at[j]], add=True)
        return carry

    lax.fori_loop(0, CH, body, 0)
    plsc.subcore_barrier()
    pltpu.sync_copy(hist.at[pl.ds(s * RPT, RPT)], out.at[c, pl.ds(s * RPT, RPT)])


SEGA = 16
NSEGA = CH // SEGA


def _sc_agg_body(g, src2d, dst2d, out, sa, sb, dseg, r0, r1, acc, s0, s1):
    c = lax.axis_index("c")
    s = lax.axis_index("s")
    bufs = (r0, r1)
    sems = (s0, s1)
    ssegs = (sa, sb)
    base = (s * NC + c) * CH

    def run_seg(t, issue_next):
        cur = ssegs[t % 2]
        nxt = ssegs[(t + 1) % 2]
        pltpu.sync_copy(dst2d.at[pl.ds(base + t * SEGA, SEGA)], dseg)
        if issue_next:
            pltpu.sync_copy(src2d.at[pl.ds(base + (t + 1) * SEGA, SEGA)], nxt)

        def pair(p, carry):
            for k in range(2):
                j = p * 2 + k
                pltpu.make_async_copy(g.at[cur.at[j]], bufs[k], sems[k]).wait()
                pltpu.sync_copy(bufs[k], acc.at[dseg.at[j]], add=True)
                pltpu.async_copy(g.at[cur.at[j + 2]], bufs[k], sems[k])
            return carry

        lax.fori_loop(0, SEGA // 2 - 1, pair, 0)
        for k in range(2):
            j = SEGA - 2 + k
            pltpu.make_async_copy(g.at[cur.at[j]], bufs[k], sems[k]).wait()
            pltpu.sync_copy(bufs[k], acc.at[dseg.at[j]], add=True)
            if issue_next:
                pltpu.async_copy(g.at[nxt.at[k]], bufs[k], sems[k])
            else:
                pltpu.async_copy(g.at[cur.at[SEGA - 1]], bufs[k], sems[k])

    def zfill(i, carry):
        r0[i // 8, pl.ds((i % 8) * 16, 16)] = jnp.zeros((16,), jnp.float32)
        return carry

    lax.fori_loop(0, 128 * 8, zfill, 0)
    for t in range(RPT // 128):
        pltpu.sync_copy(r0, acc.at[pl.ds(s * RPT + t * 128, 128)])
    pltpu.sync_copy(src2d.at[pl.ds(base, SEGA)], sa)
    plsc.subcore_barrier()

    pltpu.async_copy(g.at[sa.at[0]], r0, s0)
    pltpu.async_copy(g.at[sa.at[1]], r1, s1)

    for t in range(NSEGA):
        run_seg(t, t < NSEGA - 1)

    pltpu.make_async_copy(g.at[sa.at[0]], r0, s0).wait()
    pltpu.make_async_copy(g.at[sa.at[1]], r1, s1).wait()
    plsc.subcore_barrier()
    pltpu.sync_copy(acc.at[pl.ds(s * RPT, RPT)],
                    out.at[c, pl.ds(s * RPT, RPT)])


@functools.lru_cache(maxsize=None)
def _sc_kernels():
    mesh = plsc.VectorSubcoreMesh(**_MESH)
    sc_degree = pl.kernel(
        _sc_degree_body,
        out_type=jax.ShapeDtypeStruct((NC, NP), jnp.float32),
        mesh=mesh,
        scratch_types=[
            pltpu.VMEM((CH, 128), jnp.int32),
            pltpu.VMEM((128,), jnp.float32),
            pltpu.VMEM((RPT,), jnp.float32),
            pltpu.VMEM_SHARED((NP,), jnp.float32),
        ],
    )
    sc_agg = pl.kernel(
        _sc_agg_body,
        out_type=jax.ShapeDtypeStruct((NC, NP, D), jnp.float32),
        mesh=mesh,
        scratch_types=[
            pltpu.VMEM((SEGA, 128), jnp.int32),
            pltpu.VMEM((SEGA, 128), jnp.int32),
            pltpu.VMEM((SEGA, 128), jnp.int32),
            pltpu.VMEM((128, D), jnp.float32),
            pltpu.VMEM((128, D), jnp.float32),
            pltpu.VMEM_SHARED((NP, D), jnp.float32),
            pltpu.SemaphoreType.DMA,
            pltpu.SemaphoreType.DMA,
        ],
    )
    return sc_degree, sc_agg


def _dinv(deg_ref):
    deg = deg_ref[0, :] + deg_ref[1, :] + 1.0
    return 1.0 / jnp.sqrt(deg)


def _tc_g1(deg_ref, x_ref, w_ref, o_ref):
    dinv = _dinv(deg_ref)
    h = lax.dot_general(x_ref[...], w_ref[...], (((1,), (0,)), ((), ())),
                        precision=HP)
    o_ref[...] = h * dinv[:, None]


def _tc_mid(deg_ref, agg_ref, g_ref, w_ref, b_ref, o_ref):
    dinv = _dinv(deg_ref)
    tot = agg_ref[0] + agg_ref[1] + g_ref[...]
    h1 = jnp.maximum(tot * dinv[:, None] + b_ref[0, :][None, :], 0.0)
    h2 = lax.dot_general(h1, w_ref[...], (((1,), (0,)), ((), ())), precision=HP)
    o_ref[...] = h2 * dinv[:, None]


def _tc_final(deg_ref, agg_ref, g_ref, b2_ref, batch_ref, wv_ref, bv_ref,
              wo_ref, bo_ref, fw_ref, fb_ref, out_ref, acc, cnt):
    i = pl.program_id(0)
    dinv = _dinv(deg_ref)
    tot = agg_ref[0] + agg_ref[1] + g_ref[...]
    h = jnp.maximum(tot * dinv[:, None] + b2_ref[0, :][None, :], 0.0)
    b = batch_ref[0, 0, :]
    gi = lax.broadcasted_iota(jnp.int32, (GRAPHS, RB), 0)
    oh = (gi == b[None, :]).astype(jnp.float32)
    ps = lax.dot_general(oh, h, (((1,), (0,)), ((), ())), precision=HP)
    pc = jnp.broadcast_to(jnp.sum(oh, axis=1)[:, None], (GRAPHS, D))

    @pl.when(i == 0)
    def _():
        acc[...] = ps
        cnt[...] = pc

    @pl.when(i > 0)
    def _():
        acc[...] += ps
        cnt[...] += pc

    @pl.when(i == GRID - 1)
    def _():
        cvals = cnt[...]
        mean = acc[...] / jnp.maximum(cvals, 1.0)
        t = lax.dot_general(mean, wv_ref[...], (((1,), (0,)), ((), ())),
                            precision=HP)
        t = lax.dot_general(t, wo_ref[...], (((1,), (0,)), ((), ())),
                            precision=HP)
        bias2 = lax.dot_general(bv_ref[...], wo_ref[...],
                                (((1,), (0,)), ((), ())), precision=HP) + bo_ref[...]
        nz = jnp.where(cvals > 0.0, 1.0, 0.0)
        h2p = t + nz * bias2
        res = lax.dot_general(h2p, fw_ref[...], (((1,), (0,)), ((), ())),
                              precision=HP) + fb_ref[...]
        out_ref[...] = res


def kernel(x, edge_index, batch, W1, b1, W2, b2, in_proj_w, in_proj_b,
           out_proj_w, out_proj_b, fc_w, fc_b):
    f32 = jnp.float32
    src, dst = edge_index[0], edge_index[1]
    pad_e = EPAD - E
    pidx = jnp.arange(pad_e, dtype=jnp.int32)
    src2d = jnp.concatenate([src, pidx % N]).reshape(ROWS2D, 128)
    dst2d = jnp.concatenate([dst, N + pidx % (NP - N)]).reshape(ROWS2D, 128)
    xp = jnp.zeros((NP, D), f32).at[:N].set(x)
    batch3d = jnp.concatenate(
        [batch, jnp.full((NP - N,), GRAPHS, jnp.int32)]).reshape(GRID, 1, RB)

    _sc_degree, _sc_agg = _sc_kernels()
    degp = _sc_degree(dst2d)

    g1 = pl.pallas_call(
        _tc_g1, grid=(GRID,),
        in_specs=[pl.BlockSpec((2, RB), lambda i: (0, i)),
                  pl.BlockSpec((RB, D), lambda i: (i, 0)),
                  pl.BlockSpec((D, D), lambda i: (0, 0))],
        out_specs=pl.BlockSpec((RB, D), lambda i: (i, 0)),
        out_shape=jax.ShapeDtypeStruct((NP, D), f32),
    )(degp, xp, W1)

    agg1 = _sc_agg(g1, src2d, dst2d)

    g2 = pl.pallas_call(
        _tc_mid, grid=(GRID,),
        in_specs=[pl.BlockSpec((2, RB), lambda i: (0, i)),
                  pl.BlockSpec((2, RB, D), lambda i: (0, i, 0)),
                  pl.BlockSpec((RB, D), lambda i: (i, 0)),
                  pl.BlockSpec((D, D), lambda i: (0, 0)),
                  pl.BlockSpec((1, D), lambda i: (0, 0))],
        out_specs=pl.BlockSpec((RB, D), lambda i: (i, 0)),
        out_shape=jax.ShapeDtypeStruct((NP, D), f32),
    )(degp, agg1, g1, W2, b1.reshape(1, D))

    agg2 = _sc_agg(g2, src2d, dst2d)

    res = pl.pallas_call(
        _tc_final, grid=(GRID,),
        in_specs=[pl.BlockSpec((2, RB), lambda i: (0, i)),
                  pl.BlockSpec((2, RB, D), lambda i: (0, i, 0)),
                  pl.BlockSpec((RB, D), lambda i: (i, 0)),
                  pl.BlockSpec((1, D), lambda i: (0, 0)),
                  pl.BlockSpec((1, 1, RB), lambda i: (i, 0, 0)),
                  pl.BlockSpec((D, D), lambda i: (0, 0)),
                  pl.BlockSpec((1, D), lambda i: (0, 0)),
                  pl.BlockSpec((D, D), lambda i: (0, 0)),
                  pl.BlockSpec((1, D), lambda i: (0, 0)),
                  pl.BlockSpec((D, D), lambda i: (0, 0)),
                  pl.BlockSpec((1, D), lambda i: (0, 0))],
        out_specs=pl.BlockSpec((GRAPHS, D), lambda i: (0, 0)),
        out_shape=jax.ShapeDtypeStruct((GRAPHS, D), f32),
        scratch_shapes=[pltpu.VMEM((GRAPHS, D), f32),
                        pltpu.VMEM((GRAPHS, D), f32)],
    )(degp, agg2, g2, b2.reshape(1, D), batch3d,
      in_proj_w[2 * HID:].T, in_proj_b[2 * HID:].reshape(1, D),
      out_proj_w.T, out_proj_b.reshape(1, D),
      jnp.zeros((D, D), f32).at[:, :OUT].set(fc_w.T),
      jnp.zeros((1, D), f32).at[0, :OUT].set(fc_b))

    return res[:, :OUT]


# speedup vs baseline: 4.2233x; 1.0529x over previous
import functools

import jax
import jax.numpy as jnp
from jax import lax
from jax.experimental import pallas as pl
from jax.experimental.pallas import tpu as pltpu
from jax.experimental.pallas import tpu_sc as plsc

N = 10000
NP = 10240
D = 128
E = 320000
OUT = 10
GRAPHS = 64
HID = 128
NC, NS = 2, 16
NW = NC * NS
CH = 80
ROWS2D = NW * CH
EPAD = ROWS2D * 128
RPT = NP // NS
RB = 1024
GRID = NP // RB
HP = lax.Precision.HIGHEST

_MESH = dict(core_axis_name="c", subcore_axis_name="s",
             num_cores=NC, num_subcores=NS)


def _sc_degree_body(dst2d, out, didx, ones_v, zrow, hist):
    c = lax.axis_index("c")
    s = lax.axis_index("s")
    w = s * NC + c

    def fill_zero(i, carry):
        zrow[pl.ds(i * 16, 16)] = jnp.zeros((16,), jnp.float32)
        return carry

    lax.fori_loop(0, RPT // 16, fill_zero, 0)
    for k in range(8):
        ones_v[pl.ds(k * 16, 16)] = jnp.ones((16,), jnp.float32)
    pltpu.sync_copy(zrow, hist.at[pl.ds(s * RPT, RPT)])
    pltpu.sync_copy(dst2d.at[pl.ds(w * CH, CH)], didx)
    plsc.subcore_barrier()

    def body(j, carry):
        pltpu.sync_copy(ones_v, hist.at[didx.at[j]], add=True)
        return carry

    lax.fori_loop(0, CH, body, 0)
    plsc.subcore_barrier()
    pltpu.sync_copy(hist.at[pl.ds(s * RPT, RPT)], out.at[c, pl.ds(s * RPT, RPT)])


SEGA = 16
NSEGA = CH // SEGA


def _sc_agg_body(g, src2d, dst2d, out, sa, sb, dseg, r0, r1, acc, s0, s1):
    c = lax.axis_index("c")
    s = lax.axis_index("s")
    bufs = (r0, r1)
    sems = (s0, s1)
    ssegs = (sa, sb)
    base = (s * NC + c) * CH

    def run_seg(t, issue_next):
        cur = ssegs[t % 2]
        nxt = ssegs[(t + 1) % 2]
        pltpu.sync_copy(dst2d.at[pl.ds(base + t * SEGA, SEGA)], dseg)
        if issue_next:
            pltpu.sync_copy(src2d.at[pl.ds(base + (t + 1) * SEGA, SEGA)], nxt)

        def pair(p, carry):
            for k in range(2):
                j = p * 2 + k
                pltpu.make_async_copy(g.at[cur.at[j]], bufs[k], sems[k]).wait()
                pltpu.sync_copy(bufs[k], acc.at[dseg.at[j]], add=True)
                pltpu.async_copy(g.at[cur.at[j + 2]], bufs[k], sems[k])
            return carry

        lax.fori_loop(0, SEGA // 2 - 1, pair, 0)
        for k in range(2):
            j = SEGA - 2 + k
            pltpu.make_async_copy(g.at[cur.at[j]], bufs[k], sems[k]).wait()
            pltpu.sync_copy(bufs[k], acc.at[dseg.at[j]], add=True)
            if issue_next:
                pltpu.async_copy(g.at[nxt.at[k]], bufs[k], sems[k])
            else:
                pltpu.async_copy(g.at[cur.at[SEGA - 1]], bufs[k], sems[k])

    def zfill(i, carry):
        r0[i // 8, pl.ds((i % 8) * 16, 16)] = jnp.zeros((16,), jnp.float32)
        return carry

    lax.fori_loop(0, 128 * 8, zfill, 0)
    for t in range(RPT // 128):
        pltpu.sync_copy(r0, acc.at[pl.ds(s * RPT + t * 128, 128)])
    pltpu.sync_copy(src2d.at[pl.ds(base, SEGA)], sa)
    plsc.subcore_barrier()

    pltpu.async_copy(g.at[sa.at[0]], r0, s0)
    pltpu.async_copy(g.at[sa.at[1]], r1, s1)

    for t in range(NSEGA):
        run_seg(t, t < NSEGA - 1)

    pltpu.make_async_copy(g.at[sa.at[0]], r0, s0).wait()
    pltpu.make_async_copy(g.at[sa.at[1]], r1, s1).wait()
    plsc.subcore_barrier()
    pltpu.sync_copy(acc.at[pl.ds(s * RPT, RPT)],
                    out.at[c, pl.ds(s * RPT, RPT)])


@functools.lru_cache(maxsize=None)
def _sc_kernels():
    mesh = plsc.VectorSubcoreMesh(**_MESH)
    sc_degree = pl.kernel(
        _sc_degree_body,
        out_type=jax.ShapeDtypeStruct((NC, NP), jnp.float32),
        mesh=mesh,
        scratch_types=[
            pltpu.VMEM((CH, 128), jnp.int32),
            pltpu.VMEM((128,), jnp.float32),
            pltpu.VMEM((RPT,), jnp.float32),
            pltpu.VMEM_SHARED((NP,), jnp.float32),
        ],
    )
    sc_agg = pl.kernel(
        _sc_agg_body,
        out_type=jax.ShapeDtypeStruct((NC, NP, D), jnp.float32),
        mesh=mesh,
        scratch_types=[
            pltpu.VMEM((SEGA, 128), jnp.int32),
            pltpu.VMEM((SEGA, 128), jnp.int32),
            pltpu.VMEM((SEGA, 128), jnp.int32),
            pltpu.VMEM((128, D), jnp.float32),
            pltpu.VMEM((128, D), jnp.float32),
            pltpu.VMEM_SHARED((NP, D), jnp.float32),
            pltpu.SemaphoreType.DMA,
            pltpu.SemaphoreType.DMA,
        ],
    )
    return sc_degree, sc_agg


def _dinv(deg_ref):
    deg = deg_ref[0, :] + deg_ref[1, :] + 1.0
    return 1.0 / jnp.sqrt(deg)


def _tc_g1(deg_ref, x_ref, w_ref, o_ref):
    dinv = _dinv(deg_ref)
    h = lax.dot_general(x_ref[...], w_ref[...], (((1,), (0,)), ((), ())),
                        precision=HP)
    o_ref[...] = h * dinv[:, None]


def _tc_mid(deg_ref, agg_ref, g_ref, w_ref, b_ref, o_ref):
    dinv = _dinv(deg_ref)
    tot = agg_ref[0] + agg_ref[1] + g_ref[...]
    h1 = jnp.maximum(tot * dinv[:, None] + b_ref[0, :][None, :], 0.0)
    h2 = lax.dot_general(h1, w_ref[...], (((1,), (0,)), ((), ())), precision=HP)
    o_ref[...] = h2 * dinv[:, None]


def _tc_final(deg_ref, agg_ref, g_ref, b2_ref, batch_ref, wv_ref, bv_ref,
              wo_ref, bo_ref, fw_ref, fb_ref, out_ref, acc, cnt):
    i = pl.program_id(0)
    dinv = _dinv(deg_ref)
    tot = agg_ref[0] + agg_ref[1] + g_ref[...]
    h = jnp.maximum(tot * dinv[:, None] + b2_ref[0, :][None, :], 0.0)
    b = batch_ref[0, 0, :]
    gi = lax.broadcasted_iota(jnp.int32, (GRAPHS, RB), 0)
    oh = (gi == b[None, :]).astype(jnp.float32)
    ps = lax.dot_general(oh, h, (((1,), (0,)), ((), ())), precision=HP)
    pc = jnp.broadcast_to(jnp.sum(oh, axis=1)[:, None], (GRAPHS, D))

    @pl.when(i == 0)
    def _():
        acc[...] = ps
        cnt[...] = pc

    @pl.when(i > 0)
    def _():
        acc[...] += ps
        cnt[...] += pc

    @pl.when(i == GRID - 1)
    def _():
        cvals = cnt[...]
        mean = acc[...] / jnp.maximum(cvals, 1.0)
        t = lax.dot_general(mean, wv_ref[...], (((1,), (0,)), ((), ())),
                            precision=HP)
        t = lax.dot_general(t, wo_ref[...], (((1,), (0,)), ((), ())),
                            precision=HP)
        bias2 = lax.dot_general(bv_ref[...], wo_ref[...],
                                (((1,), (0,)), ((), ())), precision=HP) + bo_ref[...]
        nz = jnp.where(cvals > 0.0, 1.0, 0.0)
        h2p = t + nz * bias2
        res = lax.dot_general(h2p, fw_ref[...], (((1,), (0,)), ((), ())),
                              precision=HP) + fb_ref[...]
        out_ref[...] = res


def kernel(x, edge_index, batch, W1, b1, W2, b2, in_proj_w, in_proj_b,
           out_proj_w, out_proj_b, fc_w, fc_b):
    f32 = jnp.float32
    src, dst = edge_index[0], edge_index[1]
    pad_e = EPAD - E
    pidx = jnp.arange(pad_e, dtype=jnp.int32)
    src2d = jnp.concatenate([src, pidx % N]).reshape(ROWS2D, 128)
    dst2d = jnp.concatenate([dst, N + pidx % (NP - N)]).reshape(ROWS2D, 128)
    xp = jnp.zeros((NP, D), f32).at[:N].set(x)
    batch3d = jnp.concatenate(
        [batch, jnp.full((NP - N,), GRAPHS, jnp.int32)]).reshape(GRID, 1, RB)

    _sc_degree, _sc_agg = _sc_kernels()
    degp = _sc_degree(dst2d)

    g1 = pl.pallas_call(
        _tc_g1, grid=(GRID,),
        in_specs=[pl.BlockSpec((2, RB), lambda i: (0, i)),
                  pl.BlockSpec((RB, D), lambda i: (i, 0)),
                  pl.BlockSpec((D, D), lambda i: (0, 0))],
        out_specs=pl.BlockSpec((RB, D), lambda i: (i, 0)),
        out_shape=jax.ShapeDtypeStruct((NP, D), f32),
    )(degp, xp, W1)

    agg1 = _sc_agg(g1, src2d, dst2d)

    g2 = pl.pallas_call(
        _tc_mid, grid=(GRID,),
        in_specs=[pl.BlockSpec((2, RB), lambda i: (0, i)),
                  pl.BlockSpec((2, RB, D), lambda i: (0, i, 0)),
                  pl.BlockSpec((RB, D), lambda i: (i, 0)),
                  pl.BlockSpec((D, D), lambda i: (0, 0)),
                  pl.BlockSpec((1, D), lambda i: (0, 0))],
        out_specs=pl.BlockSpec((RB, D), lambda i: (i, 0)),
        out_shape=jax.ShapeDtypeStruct((NP, D), f32),
    )(degp, agg1, g1, W2, b1.reshape(1, D))

    agg2 = _sc_agg(g2, src2d, dst2d)

    res = pl.pallas_call(
        _tc_final, grid=(GRID,),
        in_specs=[pl.BlockSpec((2, RB), lambda i: (0, i)),
                  pl.BlockSpec((2, RB, D), lambda i: (0, i, 0)),
                  pl.BlockSpec((RB, D), lambda i: (i, 0)),
                  pl.BlockSpec((1, D), lambda i: (0, 0)),
                  pl.BlockSpec((1, 1, RB), lambda i: (i, 0, 0)),
                  pl.BlockSpec((D, D), lambda i: (0, 0)),
                  pl.BlockSpec((1, D), lambda i: (0, 0)),
                  pl.BlockSpec((D, D), lambda i: (0, 0)),
                  pl.BlockSpec((1, D), lambda i: (0, 0)),
                  pl.BlockSpec((D, D), lambda i: (0, 0)),
                  pl.BlockSpec((1, D), lambda i: (0, 0))],
        out_specs=pl.BlockSpec((GRAPHS, D), lambda i: (0, 0)),
        out_shape=jax.ShapeDtypeStruct((GRAPHS, D), f32),
        scratch_shapes=[pltpu.VMEM((GRAPHS, D), f32),
                        pltpu.VMEM((GRAPHS, D), f32)],
    )(degp, agg2, g2, b2.reshape(1, D), batch3d,
      in_proj_w[2 * HID:].T, in_proj_b[2 * HID:].reshape(1, D),
      out_proj_w.T, out_proj_b.reshape(1, D),
      jnp.zeros((D, D), f32).at[:, :OUT].set(fc_w.T),
      jnp.zeros((1, D), f32).at[0, :OUT].set(fc_b))

    return res[:, :OUT]


# speedup vs baseline: 4.3020x; 1.0186x over previous
import functools

import jax
import jax.numpy as jnp
from jax import lax
from jax.experimental import pallas as pl
from jax.experimental.pallas import tpu as pltpu
from jax.experimental.pallas import tpu_sc as plsc

N = 10000
NP = 10240
D = 128
E = 320000
OUT = 10
GRAPHS = 64
HID = 128
NC, NS = 2, 16
NW = NC * NS
CH = 80
ROWS2D = NW * CH
EPAD = ROWS2D * 128
RPT = NP // NS
RB = 2048
GRID = NP // RB
HP = lax.Precision.HIGHEST

_MESH = dict(core_axis_name="c", subcore_axis_name="s",
             num_cores=NC, num_subcores=NS)


def _sc_degree_body(dst2d, out, didx, ones_v, zrow, hist):
    c = lax.axis_index("c")
    s = lax.axis_index("s")
    w = s * NC + c

    def fill_zero(i, carry):
        zrow[pl.ds(i * 16, 16)] = jnp.zeros((16,), jnp.float32)
        return carry

    lax.fori_loop(0, RPT // 16, fill_zero, 0)
    for k in range(8):
        ones_v[pl.ds(k * 16, 16)] = jnp.ones((16,), jnp.float32)
    pltpu.sync_copy(zrow, hist.at[pl.ds(s * RPT, RPT)])
    pltpu.sync_copy(dst2d.at[pl.ds(w * CH, CH)], didx)
    plsc.subcore_barrier()

    def body(j, carry):
        pltpu.sync_copy(ones_v, hist.at[didx.at[j]], add=True)
        return carry

    lax.fori_loop(0, CH, body, 0)
    plsc.subcore_barrier()
    pltpu.sync_copy(hist.at[pl.ds(s * RPT, RPT)], out.at[c, pl.ds(s * RPT, RPT)])


SEGA = 16
NSEGA = CH // SEGA


def _sc_agg_body(g, src2d, dst2d, out, sa, sb, dseg, r0, r1, acc, s0, s1):
    c = lax.axis_index("c")
    s = lax.axis_index("s")
    bufs = (r0, r1)
    sems = (s0, s1)
    ssegs = (sa, sb)
    base = (s * NC + c) * CH

    def run_seg(t, issue_next):
        cur = ssegs[t % 2]
        nxt = ssegs[(t + 1) % 2]
        pltpu.sync_copy(dst2d.at[pl.ds(base + t * SEGA, SEGA)], dseg)
        if issue_next:
            pltpu.sync_copy(src2d.at[pl.ds(base + (t + 1) * SEGA, SEGA)], nxt)

        def pair(p, carry):
            for k in range(2):
                j = p * 2 + k
                pltpu.make_async_copy(g.at[cur.at[j]], bufs[k], sems[k]).wait()
                pltpu.sync_copy(bufs[k], acc.at[dseg.at[j]], add=True)
                pltpu.async_copy(g.at[cur.at[j + 2]], bufs[k], sems[k])
            return carry

        lax.fori_loop(0, SEGA // 2 - 1, pair, 0)
        for k in range(2):
            j = SEGA - 2 + k
            pltpu.make_async_copy(g.at[cur.at[j]], bufs[k], sems[k]).wait()
            pltpu.sync_copy(bufs[k], acc.at[dseg.at[j]], add=True)
            if issue_next:
                pltpu.async_copy(g.at[nxt.at[k]], bufs[k], sems[k])
            else:
                pltpu.async_copy(g.at[cur.at[SEGA - 1]], bufs[k], sems[k])

    def zfill(i, carry):
        r0[i // 8, pl.ds((i % 8) * 16, 16)] = jnp.zeros((16,), jnp.float32)
        return carry

    lax.fori_loop(0, 128 * 8, zfill, 0)
    for t in range(RPT // 128):
        pltpu.sync_copy(r0, acc.at[pl.ds(s * RPT + t * 128, 128)])
    pltpu.sync_copy(src2d.at[pl.ds(base, SEGA)], sa)
    plsc.subcore_barrier()

    pltpu.async_copy(g.at[sa.at[0]], r0, s0)
    pltpu.async_copy(g.at[sa.at[1]], r1, s1)

    for t in range(NSEGA):
        run_seg(t, t < NSEGA - 1)

    pltpu.make_async_copy(g.at[sa.at[0]], r0, s0).wait()
    pltpu.make_async_copy(g.at[sa.at[1]], r1, s1).wait()
    plsc.subcore_barrier()
    pltpu.sync_copy(acc.at[pl.ds(s * RPT, RPT)],
                    out.at[c, pl.ds(s * RPT, RPT)])


@functools.lru_cache(maxsize=None)
def _sc_kernels():
    mesh = plsc.VectorSubcoreMesh(**_MESH)
    sc_degree = pl.kernel(
        _sc_degree_body,
        out_type=jax.ShapeDtypeStruct((NC, NP), jnp.float32),
        mesh=mesh,
        scratch_types=[
            pltpu.VMEM((CH, 128), jnp.int32),
            pltpu.VMEM((128,), jnp.float32),
            pltpu.VMEM((RPT,), jnp.float32),
            pltpu.VMEM_SHARED((NP,), jnp.float32),
        ],
    )
    sc_agg = pl.kernel(
        _sc_agg_body,
        out_type=jax.ShapeDtypeStruct((NC, NP, D), jnp.float32),
        mesh=mesh,
        scratch_types=[
            pltpu.VMEM((SEGA, 128), jnp.int32),
            pltpu.VMEM((SEGA, 128), jnp.int32),
            pltpu.VMEM((SEGA, 128), jnp.int32),
            pltpu.VMEM((128, D), jnp.float32),
            pltpu.VMEM((128, D), jnp.float32),
            pltpu.VMEM_SHARED((NP, D), jnp.float32),
            pltpu.SemaphoreType.DMA,
            pltpu.SemaphoreType.DMA,
        ],
    )
    return sc_degree, sc_agg


def _dinv(deg_ref):
    deg = deg_ref[0, :] + deg_ref[1, :] + 1.0
    return 1.0 / jnp.sqrt(deg)


def _tc_g1(deg_ref, x_ref, w_ref, o_ref):
    dinv = _dinv(deg_ref)
    h = lax.dot_general(x_ref[...], w_ref[...], (((1,), (0,)), ((), ())),
                        precision=HP)
    o_ref[...] = h * dinv[:, None]


def _tc_mid(deg_ref, agg_ref, g_ref, w_ref, b_ref, o_ref):
    dinv = _dinv(deg_ref)
    tot = agg_ref[0] + agg_ref[1] + g_ref[...]
    h1 = jnp.maximum(tot * dinv[:, None] + b_ref[0, :][None, :], 0.0)
    h2 = lax.dot_general(h1, w_ref[...], (((1,), (0,)), ((), ())), precision=HP)
    o_ref[...] = h2 * dinv[:, None]


def _tc_final(deg_ref, agg_ref, g_ref, b2_ref, batch_ref, wv_ref, bv_ref,
              wo_ref, bo_ref, fw_ref, fb_ref, out_ref, acc, cnt):
    i = pl.program_id(0)
    dinv = _dinv(deg_ref)
    tot = agg_ref[0] + agg_ref[1] + g_ref[...]
    h = jnp.maximum(tot * dinv[:, None] + b2_ref[0, :][None, :], 0.0)
    b = batch_ref[0, 0, :]
    gi = lax.broadcasted_iota(jnp.int32, (GRAPHS, RB), 0)
    oh = (gi == b[None, :]).astype(jnp.float32)
    ps = lax.dot_general(oh, h, (((1,), (0,)), ((), ())), precision=HP)
    pc = jnp.broadcast_to(jnp.sum(oh, axis=1)[:, None], (GRAPHS, D))

    @pl.when(i == 0)
    def _():
        acc[...] = ps
        cnt[...] = pc

    @pl.when(i > 0)
    def _():
        acc[...] += ps
        cnt[...] += pc

    @pl.when(i == GRID - 1)
    def _():
        cvals = cnt[...]
        mean = acc[...] / jnp.maximum(cvals, 1.0)
        t = lax.dot_general(mean, wv_ref[...], (((1,), (0,)), ((), ())),
                            precision=HP)
        t = lax.dot_general(t, wo_ref[...], (((1,), (0,)), ((), ())),
                            precision=HP)
        bias2 = lax.dot_general(bv_ref[...], wo_ref[...],
                                (((1,), (0,)), ((), ())), precision=HP) + bo_ref[...]
        nz = jnp.where(cvals > 0.0, 1.0, 0.0)
        h2p = t + nz * bias2
        res = lax.dot_general(h2p, fw_ref[...], (((1,), (0,)), ((), ())),
                              precision=HP) + fb_ref[...]
        out_ref[...] = res


def kernel(x, edge_index, batch, W1, b1, W2, b2, in_proj_w, in_proj_b,
           out_proj_w, out_proj_b, fc_w, fc_b):
    f32 = jnp.float32
    src, dst = edge_index[0], edge_index[1]
    pad_e = EPAD - E
    pidx = jnp.arange(pad_e, dtype=jnp.int32)
    src2d = jnp.concatenate([src, pidx % N]).reshape(ROWS2D, 128)
    dst2d = jnp.concatenate([dst, N + pidx % (NP - N)]).reshape(ROWS2D, 128)
    xp = jnp.zeros((NP, D), f32).at[:N].set(x)
    batch3d = jnp.concatenate(
        [batch, jnp.full((NP - N,), GRAPHS, jnp.int32)]).reshape(GRID, 1, RB)

    _sc_degree, _sc_agg = _sc_kernels()
    degp = _sc_degree(dst2d)

    g1 = pl.pallas_call(
        _tc_g1, grid=(GRID,),
        in_specs=[pl.BlockSpec((2, RB), lambda i: (0, i)),
                  pl.BlockSpec((RB, D), lambda i: (i, 0)),
                  pl.BlockSpec((D, D), lambda i: (0, 0))],
        out_specs=pl.BlockSpec((RB, D), lambda i: (i, 0)),
        out_shape=jax.ShapeDtypeStruct((NP, D), f32),
    )(degp, xp, W1)

    agg1 = _sc_agg(g1, src2d, dst2d)

    g2 = pl.pallas_call(
        _tc_mid, grid=(GRID,),
        in_specs=[pl.BlockSpec((2, RB), lambda i: (0, i)),
                  pl.BlockSpec((2, RB, D), lambda i: (0, i, 0)),
                  pl.BlockSpec((RB, D), lambda i: (i, 0)),
                  pl.BlockSpec((D, D), lambda i: (0, 0)),
                  pl.BlockSpec((1, D), lambda i: (0, 0))],
        out_specs=pl.BlockSpec((RB, D), lambda i: (i, 0)),
        out_shape=jax.ShapeDtypeStruct((NP, D), f32),
    )(degp, agg1, g1, W2, b1.reshape(1, D))

    agg2 = _sc_agg(g2, src2d, dst2d)

    res = pl.pallas_call(
        _tc_final, grid=(GRID,),
        in_specs=[pl.BlockSpec((2, RB), lambda i: (0, i)),
                  pl.BlockSpec((2, RB, D), lambda i: (0, i, 0)),
                  pl.BlockSpec((RB, D), lambda i: (i, 0)),
                  pl.BlockSpec((1, D), lambda i: (0, 0)),
                  pl.BlockSpec((1, 1, RB), lambda i: (i, 0, 0)),
                  pl.BlockSpec((D, D), lambda i: (0, 0)),
                  pl.BlockSpec((1, D), lambda i: (0, 0)),
                  pl.BlockSpec((D, D), lambda i: (0, 0)),
                  pl.BlockSpec((1, D), lambda i: (0, 0)),
                  pl.BlockSpec((D, D), lambda i: (0, 0)),
                  pl.BlockSpec((1, D), lambda i: (0, 0))],
        out_specs=pl.BlockSpec((GRAPHS, D), lambda i: (0, 0)),
        out_shape=jax.ShapeDtypeStruct((GRAPHS, D), f32),
        scratch_shapes=[pltpu.VMEM((GRAPHS, D), f32),
                        pltpu.VMEM((GRAPHS, D), f32)],
    )(degp, agg2, g2, b2.reshape(1, D), batch3d,
      in_proj_w[2 * HID:].T, in_proj_b[2 * HID:].reshape(1, D),
      out_proj_w.T, out_proj_b.reshape(1, D),
      jnp.zeros((D, D), f32).at[:, :OUT].set(fc_w.T),
      jnp.zeros((1, D), f32).at[0, :OUT].set(fc_b))

    return res[:, :OUT]
